# baseline (speedup 1.0000x reference)
import jax
import jax.numpy as jnp
from jax.experimental import pallas as pl
from jax.experimental.pallas import tpu as pltpu

_D = 2048
_N_TOK = 8192
_TOTAL = _D * _D * 2 + _D * 2
_K = max(1, int(0.05 * _TOTAL))
_TOPBIT = -2147483648


def _pat_to_f32(p):
    b = jnp.where(p < 0, p ^ jnp.int32(_TOPBIT), jnp.bitwise_not(p))
    return jax.lax.bitcast_convert_type(jnp.full((1, 1), b, jnp.int32),
                                        jnp.float32)


def _select_body(s_hh_ref, s_bhh_ref, s_out_ref, s_bout_ref, bits_ref):
    def count_ge(c):
        n = jnp.sum((s_hh_ref[...] >= c).astype(jnp.int32))
        n += jnp.sum((s_out_ref[...] >= c).astype(jnp.int32))
        n += jnp.sum((s_bhh_ref[...] >= c).astype(jnp.int32))
        n += jnp.sum((s_bout_ref[...] >= c).astype(jnp.int32))
        return n

    def step(i, p):
        shift = 30 - 2 * i
        hits = jnp.int32(0)
        for d in (1, 2, 3):
            cand = p | (jnp.int32(d) << shift)
            cnt = count_ge(_pat_to_f32(cand))
            hits += (cnt >= _K).astype(jnp.int32)
        return p | (hits << shift)

    p = jax.lax.fori_loop(0, 16, step, jnp.int32(0))
    bits_ref[0, 0] = jnp.where(p < 0, p ^ jnp.int32(_TOPBIT),
                               jnp.bitwise_not(p))


def _select_threshold(s_hh, s_bhh, s_out, s_bout):
    vmem = lambda: pl.BlockSpec(memory_space=pltpu.VMEM)
    return pl.pallas_call(
        _select_body,
        in_specs=[vmem(), vmem(), vmem(), vmem()],
        out_specs=pl.BlockSpec(memory_space=pltpu.SMEM),
        out_shape=jax.ShapeDtypeStruct((1, 1), jnp.int32),
    )(s_hh, s_bhh.reshape(8, _D // 8), s_out, s_bout.reshape(8, _D // 8))


_NCHUNK = 8
_ROWS = _D // _NCHUNK


def _mask_body(thr_ref, wihT_ref, whhT_ref, shhT_ref, woutT_ref, soutT_ref,
               bhh_ref, sbhh_ref, bout_ref, sbout_ref,
               wihTb_ref, wmhhT_ref, wmoutT_ref, bmhh_ref, bmout_ref):
    thr = thr_ref[0, 0]
    wihTb_ref[...] = wihT_ref[...].astype(jnp.bfloat16)
    wmhhT_ref[...] = (whhT_ref[...] * (shhT_ref[...] >= thr)
                      ).astype(jnp.bfloat16)
    wmoutT_ref[...] = (woutT_ref[...] * (soutT_ref[...] >= thr)
                       ).astype(jnp.bfloat16)
    bmhh_ref[...] = bhh_ref[...] * (sbhh_ref[...] >= thr).astype(jnp.float32)
    bmout_ref[...] = bout_ref[...] * (sbout_ref[...] >= thr).astype(jnp.float32)


def _mask_weights(thr, wihT, whhT, shhT, woutT, soutT, bhh, sbhh, bout, sbout):
    big = pl.BlockSpec((_ROWS, _D), lambda c: (c, 0))
    vec = pl.BlockSpec((1, _D), lambda c: (0, 0))
    return pl.pallas_call(
        _mask_body,
        grid=(_NCHUNK,),
        in_specs=[pl.BlockSpec(memory_space=pltpu.SMEM),
                  big, big, big, big, big, vec, vec, vec, vec],
        out_specs=[big, big, big, vec, vec],
        out_shape=[jax.ShapeDtypeStruct((_D, _D), jnp.bfloat16),
                   jax.ShapeDtypeStruct((_D, _D), jnp.bfloat16),
                   jax.ShapeDtypeStruct((_D, _D), jnp.bfloat16),
                   jax.ShapeDtypeStruct((1, _D), jnp.float32),
                   jax.ShapeDtypeStruct((1, _D), jnp.float32)],
    )(thr, wihT, whhT, shhT, woutT, soutT, bhh, sbhh, bout, sbout)


_TILE_M = 512


def _dot_nt(a, b):
    return jax.lax.dot_general(a, b, (((1,), (1,)), ((), ())),
                               preferred_element_type=jnp.float32)


def _fwd_body(x_ref, wih_ref, wmhh_ref, bmhh_ref, wmout_ref, bmout_ref,
              out_ref):
    h = jnp.tanh(_dot_nt(x_ref[...].astype(jnp.bfloat16), wih_ref[...]))
    h2 = jnp.tanh(_dot_nt(h.astype(jnp.bfloat16), wmhh_ref[...])
                  + bmhh_ref[...])
    out_ref[...] = _dot_nt(h2.astype(jnp.bfloat16), wmout_ref[...]) \
        + bmout_ref[...]


def _forward(x, wihT, wmhhT, bmhh, wmoutT, bmout):
    xspec = pl.BlockSpec((_TILE_M, _D), lambda m: (m, 0))
    wspec = pl.BlockSpec((_D, _D), lambda m: (0, 0))
    vec = pl.BlockSpec((1, _D), lambda m: (0, 0))
    return pl.pallas_call(
        _fwd_body,
        grid=(_N_TOK // _TILE_M,),
        in_specs=[xspec, wspec, wspec, vec, wspec, vec],
        out_specs=xspec,
        out_shape=jax.ShapeDtypeStruct((_N_TOK, _D), jnp.float32),
    )(x, wihT, wmhhT, bmhh, wmoutT, bmout)


def kernel(x, W_ih, W_hh, b_hh, W_out, b_out, s_hh, s_b_hh, s_out, s_b_out):
    bits = _select_threshold(s_hh, s_b_hh, s_out, s_b_out)
    thr = jax.lax.bitcast_convert_type(bits, jnp.float32)
    wihb, wmhh, wmout, bmhh, bmout = _mask_weights(
        thr, W_ih, W_hh, s_hh, W_out, s_out,
        b_hh.reshape(1, _D), s_b_hh.reshape(1, _D),
        b_out.reshape(1, _D), s_b_out.reshape(1, _D))
    return _forward(x, wihb, wmhh, bmhh, wmout, bmout)
